# baseline (speedup 1.0000x reference)
import functools
import math

import jax
import jax.numpy as jnp
from jax import lax
from jax.experimental import pallas as pl
from jax.experimental.pallas import tpu as pltpu
from jax.experimental.pallas import tpu_sc as plsc

_C = 100000
_E = 128
_B = 1024
_SCALE = 64.0
_LOG2E = 1.4426950408889634
_LN2 = 0.6931471805599453
_K2 = _SCALE * _LOG2E
_COS_M = math.cos(0.5)
_SIN_M = math.sin(0.5)
_EPS = 1e-12
_CHUNK = 2000
_STEPS = _C // (2 * _CHUNK)

_NW = 32
_BPW = _B // _NW


def _gather_rows(weight, labels):
    mesh = plsc.VectorSubcoreMesh(core_axis_name="c", subcore_axis_name="s")

    @functools.partial(
        pl.kernel,
        mesh=mesh,
        out_type=jax.ShapeDtypeStruct((_B, _E), jnp.float32),
        scratch_types=[
            pltpu.VMEM((_BPW,), jnp.int32),
            pltpu.VMEM((_BPW, _E), jnp.float32),
            pltpu.SemaphoreType.DMA,
        ],
    )
    def k(table_hbm, idx_hbm, out_hbm, idx_v, rows_v, sem):
        wid = lax.axis_index("s") * 2 + lax.axis_index("c")
        base = wid * _BPW
        pltpu.sync_copy(idx_hbm.at[pl.ds(base, _BPW)], idx_v)
        pltpu.async_copy(table_hbm.at[idx_v], rows_v, sem).wait()
        pltpu.sync_copy(rows_v, out_hbm.at[pl.ds(base, _BPW)])

    return k(weight, labels)


def _fused_body(emb_ref, wa_ref, wb_ref, wt_ref, loss_ref, sorted_ref,
                emb_n_ref, zsum_ref):
    i = pl.program_id(0)

    @pl.when(i == 0)
    def _init():
        e = emb_ref[...]
        norm = jnp.sqrt(jnp.sum(e * e, axis=1, keepdims=True))
        emb_n_ref[...] = e * (_K2 / jnp.maximum(norm, _EPS))
        zsum_ref[...] = jnp.zeros(zsum_ref.shape, jnp.float32)

    acc = zsum_ref[...]
    for w_ref in (wa_ref, wb_ref):
        w = w_ref[...]
        ssrow = lax.dot_general(
            jnp.ones((1, _E), jnp.float32), w * w, (((1,), (1,)), ((), ())),
            preferred_element_type=jnp.float32,
            precision=lax.Precision.HIGHEST,
        )
        rn = lax.rsqrt(jnp.maximum(ssrow, _EPS * _EPS))
        s2u = lax.dot_general(
            emb_n_ref[...], w, (((1,), (1,)), ((), ())),
            preferred_element_type=jnp.float32,
        )
        terms = jnp.exp2(jnp.clip(s2u * rn, -_K2, _K2))
        acc = acc + jnp.sum(terms, axis=1, keepdims=True)
    zsum_ref[...] = acc

    @pl.when(i == _STEPS - 1)
    def _final():
        en = emb_n_ref[...] * (1.0 / _K2)
        wt = wt_ref[...]
        wtn = wt / jnp.maximum(
            jnp.sqrt(jnp.sum(wt * wt, axis=1, keepdims=True)), _EPS)
        t = jnp.sum(en * wtn, axis=1, keepdims=True)
        t_c = jnp.clip(t, -1.0, 1.0)
        s_orig2 = t_c * _K2
        t_cc = jnp.clip(t_c, -1.0 + 1e-7, 1.0 - 1e-7)
        s_mod2 = (t_cc * _COS_M - jnp.sqrt(1.0 - t_cc * t_cc) * _SIN_M) * _K2
        z = zsum_ref[...] - jnp.exp2(s_orig2) + jnp.exp2(s_mod2)
        z = jnp.maximum(z, 1e-37)
        lp2 = s_mod2 - jnp.log2(z)
        pos = jnp.exp2(lp2)
        lnpos = jnp.maximum(lp2 * _LN2, math.log(1e-30))
        loss_ref[...] = -jnp.sum(lnpos, axis=(0, 1), keepdims=True) / _B

        col = lax.broadcasted_iota(jnp.int32, (_B, _B), 1)
        row = lax.broadcasted_iota(jnp.int32, (_B, _B), 0)
        diagm = jnp.where(row == col, pos, 0.0)
        vrow = lax.dot_general(
            jnp.ones((1, _B), jnp.float32), diagm, (((1,), (0,)), ((), ())),
            preferred_element_type=jnp.float32,
            precision=lax.Precision.HIGHEST,
        )
        gt = (vrow > pos) | ((vrow == pos) & (col < row))
        rank = jnp.sum(gt.astype(jnp.float32), axis=1, keepdims=True)
        hit = rank == col.astype(jnp.float32)
        sorted_ref[...] = jnp.sum(jnp.where(hit, pos, 0.0), axis=0,
                                  keepdims=True)


def _fused(emb, weight, wt):
    return pl.pallas_call(
        _fused_body,
        grid=(_STEPS,),
        in_specs=[
            pl.BlockSpec((_B, _E), lambda i: (0, 0)),
            pl.BlockSpec((_CHUNK, _E), lambda i: (i, 0)),
            pl.BlockSpec((_CHUNK, _E), lambda i: (i + _STEPS, 0)),
            pl.BlockSpec((_B, _E), lambda i: (0, 0)),
        ],
        out_specs=[
            pl.BlockSpec((1, 1), lambda i: (0, 0)),
            pl.BlockSpec((1, _B), lambda i: (0, 0)),
        ],
        out_shape=[
            jax.ShapeDtypeStruct((1, 1), jnp.float32),
            jax.ShapeDtypeStruct((1, _B), jnp.float32),
        ],
        scratch_shapes=[
            pltpu.VMEM((_B, _E), jnp.float32),
            pltpu.VMEM((_B, 1), jnp.float32),
        ],
    )(emb, weight, weight, wt)


def kernel(local_embeddings, local_labels, weight):
    wt = _gather_rows(weight, local_labels.astype(jnp.int32))
    loss, sorted_row = _fused(local_embeddings, weight, wt)
    return loss[0, 0], sorted_row.reshape(_B)


# speedup vs baseline: 1.3214x; 1.3214x over previous
import functools
import math

import jax
import jax.numpy as jnp
from jax import lax
from jax.experimental import pallas as pl
from jax.experimental.pallas import tpu as pltpu
from jax.experimental.pallas import tpu_sc as plsc

_C = 100000
_E = 128
_B = 1024
_SCALE = 64.0
_LOG2E = 1.4426950408889634
_LN2 = 0.6931471805599453
_K2 = _SCALE * _LOG2E
_COS_M = math.cos(0.5)
_SIN_M = math.sin(0.5)
_EPS = 1e-12
_CHUNK = 2000
_STEPS = _C // (2 * _CHUNK)

_NW = 32
_BPW = _B // _NW


def _gather_rows(weight, labels):
    mesh = plsc.VectorSubcoreMesh(core_axis_name="c", subcore_axis_name="s")

    @functools.partial(
        pl.kernel,
        mesh=mesh,
        out_type=jax.ShapeDtypeStruct((_B, _E), jnp.float32),
        scratch_types=[
            pltpu.VMEM((_BPW,), jnp.int32),
            pltpu.VMEM((_BPW, _E), jnp.float32),
            pltpu.SemaphoreType.DMA,
        ],
    )
    def k(table_hbm, idx_hbm, out_hbm, idx_v, rows_v, sem):
        wid = lax.axis_index("s") * 2 + lax.axis_index("c")
        base = wid * _BPW
        pltpu.sync_copy(idx_hbm.at[pl.ds(base, _BPW)], idx_v)
        pltpu.async_copy(table_hbm.at[idx_v], rows_v, sem).wait()
        pltpu.sync_copy(rows_v, out_hbm.at[pl.ds(base, _BPW)])

    return k(weight, labels)


def _fused_body(emb_ref, wa_ref, wb_ref, wt_ref, loss_ref, sorted_ref,
                emb_n_ref, zsum_ref):
    i = pl.program_id(0)

    @pl.when(i == 0)
    def _init():
        e = emb_ref[...]
        norm = jnp.sqrt(jnp.sum(e * e, axis=1, keepdims=True))
        emb_n_ref[...] = e * (_K2 / jnp.maximum(norm, _EPS))
        zsum_ref[...] = jnp.zeros(zsum_ref.shape, jnp.float32)

    acc = zsum_ref[...]
    for w_ref in (wa_ref, wb_ref):
        w = w_ref[...]
        ssrow = lax.dot_general(
            jnp.ones((1, _E), jnp.float32), w * w, (((1,), (1,)), ((), ())),
            preferred_element_type=jnp.float32,
        )
        rn = lax.rsqrt(jnp.maximum(ssrow, _EPS * _EPS))
        s2u = lax.dot_general(
            emb_n_ref[...], w, (((1,), (1,)), ((), ())),
            preferred_element_type=jnp.float32,
        )
        terms = jnp.exp2(jnp.clip(s2u * rn, -_K2, _K2))
        acc = acc + jnp.sum(terms, axis=1, keepdims=True)
    zsum_ref[...] = acc

    @pl.when(i == _STEPS - 1)
    def _final():
        en = emb_n_ref[...] * (1.0 / _K2)
        wt = wt_ref[...]
        wtn = wt / jnp.maximum(
            jnp.sqrt(jnp.sum(wt * wt, axis=1, keepdims=True)), _EPS)
        t = jnp.sum(en * wtn, axis=1, keepdims=True)
        t_c = jnp.clip(t, -1.0, 1.0)
        s_orig2 = t_c * _K2
        t_cc = jnp.clip(t_c, -1.0 + 1e-7, 1.0 - 1e-7)
        s_mod2 = (t_cc * _COS_M - jnp.sqrt(1.0 - t_cc * t_cc) * _SIN_M) * _K2
        z = zsum_ref[...] - jnp.exp2(s_orig2) + jnp.exp2(s_mod2)
        z = jnp.maximum(z, 1e-37)
        lp2 = s_mod2 - jnp.log2(z)
        pos = jnp.exp2(lp2)
        lnpos = jnp.maximum(lp2 * _LN2, math.log(1e-30))
        loss_ref[...] = -jnp.sum(lnpos, axis=(0, 1), keepdims=True) / _B

        col = lax.broadcasted_iota(jnp.int32, (_B, _B), 1)
        row = lax.broadcasted_iota(jnp.int32, (_B, _B), 0)
        diagm = jnp.where(row == col, pos, 0.0)
        vrow = lax.dot_general(
            jnp.ones((1, _B), jnp.float32), diagm, (((1,), (0,)), ((), ())),
            preferred_element_type=jnp.float32,
            precision=lax.Precision.HIGHEST,
        )
        gt = (vrow > pos) | ((vrow == pos) & (col < row))
        rank = jnp.sum(gt.astype(jnp.float32), axis=1, keepdims=True)
        hit = rank == col.astype(jnp.float32)
        sorted_ref[...] = jnp.sum(jnp.where(hit, pos, 0.0), axis=0,
                                  keepdims=True)


def _fused(emb, weight, wt):
    return pl.pallas_call(
        _fused_body,
        grid=(_STEPS,),
        in_specs=[
            pl.BlockSpec((_B, _E), lambda i: (0, 0)),
            pl.BlockSpec((_CHUNK, _E), lambda i: (i, 0)),
            pl.BlockSpec((_CHUNK, _E), lambda i: (i + _STEPS, 0)),
            pl.BlockSpec((_B, _E), lambda i: (0, 0)),
        ],
        out_specs=[
            pl.BlockSpec((1, 1), lambda i: (0, 0)),
            pl.BlockSpec((1, _B), lambda i: (0, 0)),
        ],
        out_shape=[
            jax.ShapeDtypeStruct((1, 1), jnp.float32),
            jax.ShapeDtypeStruct((1, _B), jnp.float32),
        ],
        scratch_shapes=[
            pltpu.VMEM((_B, _E), jnp.float32),
            pltpu.VMEM((_B, 1), jnp.float32),
        ],
    )(emb, weight, weight, wt)


def kernel(local_embeddings, local_labels, weight):
    wt = _gather_rows(weight, local_labels.astype(jnp.int32))
    loss, sorted_row = _fused(local_embeddings, weight, wt)
    return loss[0, 0], sorted_row.reshape(_B)


# speedup vs baseline: 1.3919x; 1.0534x over previous
import functools
import math

import jax
import jax.numpy as jnp
from jax import lax
from jax.experimental import pallas as pl
from jax.experimental.pallas import tpu as pltpu
from jax.experimental.pallas import tpu_sc as plsc

_C = 100000
_E = 128
_B = 1024
_SCALE = 64.0
_LOG2E = 1.4426950408889634
_LN2 = 0.6931471805599453
_K2 = _SCALE * _LOG2E
_COS_M = math.cos(0.5)
_SIN_M = math.sin(0.5)
_EPS = 1e-12
_CHUNK = 2000
_STEPS = _C // (2 * _CHUNK)

_NW = 32
_BPW = _B // _NW


def _gather_rows(weight, labels):
    mesh = plsc.VectorSubcoreMesh(core_axis_name="c", subcore_axis_name="s")

    @functools.partial(
        pl.kernel,
        mesh=mesh,
        out_type=jax.ShapeDtypeStruct((_B, _E), jnp.float32),
        scratch_types=[
            pltpu.VMEM((_BPW,), jnp.int32),
            pltpu.VMEM((_BPW, _E), jnp.float32),
            pltpu.SemaphoreType.DMA,
        ],
    )
    def k(table_hbm, idx_hbm, out_hbm, idx_v, rows_v, sem):
        wid = lax.axis_index("s") * 2 + lax.axis_index("c")
        base = wid * _BPW
        pltpu.sync_copy(idx_hbm.at[pl.ds(base, _BPW)], idx_v)
        pltpu.async_copy(table_hbm.at[idx_v], rows_v, sem).wait()
        pltpu.sync_copy(rows_v, out_hbm.at[pl.ds(base, _BPW)])

    return k(weight, labels)


def _stats_body(emb_ref, wa_ref, wb_ref, zsum_ref, emb_n_ref):
    i = pl.program_id(0)

    @pl.when(i == 0)
    def _init():
        e = emb_ref[...]
        norm = jnp.sqrt(jnp.sum(e * e, axis=1, keepdims=True))
        emb_n_ref[...] = e / jnp.maximum(norm, _EPS)
        zsum_ref[...] = jnp.zeros(zsum_ref.shape, jnp.float32)

    acc = zsum_ref[...]
    for w_ref in (wa_ref, wb_ref):
        w = w_ref[...]
        ss = jnp.sum(w * w, axis=1, keepdims=True)
        wn = w * (_K2 * lax.rsqrt(jnp.maximum(ss, _EPS * _EPS)))
        s2 = lax.dot_general(
            emb_n_ref[...], wn, (((1,), (1,)), ((), ())),
            preferred_element_type=jnp.float32,
        )
        terms = jnp.exp2(jnp.clip(s2, -_K2, _K2))
        acc = acc + jnp.sum(terms, axis=1, keepdims=True)
    zsum_ref[...] = acc


def _zsum(emb, weight):
    return pl.pallas_call(
        _stats_body,
        grid=(_STEPS,),
        in_specs=[
            pl.BlockSpec((_B, _E), lambda i: (0, 0)),
            pl.BlockSpec((_CHUNK, _E), lambda i: (i, 0)),
            pl.BlockSpec((_CHUNK, _E), lambda i: (i + _STEPS, 0)),
        ],
        out_specs=pl.BlockSpec((_B, 1), lambda i: (0, 0)),
        out_shape=jax.ShapeDtypeStruct((_B, 1), jnp.float32),
        scratch_shapes=[pltpu.VMEM((_B, _E), jnp.float32)],
    )(emb, weight, weight)


def _finalize_body(emb_ref, wt_ref, zsum_ref, loss_ref, sorted_ref):
    e = emb_ref[...]
    en = e / jnp.maximum(jnp.sqrt(jnp.sum(e * e, axis=1, keepdims=True)), _EPS)
    wt = wt_ref[...]
    wtn = wt / jnp.maximum(
        jnp.sqrt(jnp.sum(wt * wt, axis=1, keepdims=True)), _EPS)
    t = jnp.sum(en * wtn, axis=1, keepdims=True)
    t_c = jnp.clip(t, -1.0, 1.0)
    s_orig2 = t_c * _K2
    t_cc = jnp.clip(t_c, -1.0 + 1e-7, 1.0 - 1e-7)
    s_mod2 = (t_cc * _COS_M - jnp.sqrt(1.0 - t_cc * t_cc) * _SIN_M) * _K2
    z = zsum_ref[...] - jnp.exp2(s_orig2) + jnp.exp2(s_mod2)
    z = jnp.maximum(z, 1e-37)
    lp2 = s_mod2 - jnp.log2(z)
    pos = jnp.exp2(lp2)
    lnpos = jnp.maximum(lp2 * _LN2, math.log(1e-30))
    loss_ref[...] = -jnp.sum(lnpos, axis=(0, 1), keepdims=True) / _B

    col = lax.broadcasted_iota(jnp.int32, (_B, _B), 1)
    row = lax.broadcasted_iota(jnp.int32, (_B, _B), 0)
    diagm = jnp.where(row == col, pos, 0.0)
    vrow = lax.dot_general(
        jnp.ones((1, _B), jnp.float32), diagm, (((1,), (0,)), ((), ())),
        preferred_element_type=jnp.float32,
    )
    gt = (vrow > pos) | ((vrow == pos) & (col < row))
    rank = jnp.sum(gt.astype(jnp.float32), axis=1, keepdims=True)
    hit = rank == col.astype(jnp.float32)
    sorted_ref[...] = jnp.sum(jnp.where(hit, pos, 0.0), axis=0, keepdims=True)


def _finalize(emb, wt, zsum):
    return pl.pallas_call(
        _finalize_body,
        out_shape=[
            jax.ShapeDtypeStruct((1, 1), jnp.float32),
            jax.ShapeDtypeStruct((1, _B), jnp.float32),
        ],
    )(emb, wt, zsum)


def kernel(local_embeddings, local_labels, weight):
    wt = _gather_rows(weight, local_labels.astype(jnp.int32))
    zsum = _zsum(local_embeddings, weight)
    loss, sorted_row = _finalize(local_embeddings, wt, zsum)
    return loss[0, 0], sorted_row.reshape(_B)
